# baseline (speedup 1.0000x reference)
import jax
import jax.numpy as jnp
from jax.experimental import pallas as pl
from jax.experimental.pallas import tpu as pltpu

N = 10000
K = 256
D = 128
BK = 128
BN = 2000
P1 = K // BK
NH = N // 2


def _fused_body(
    tar_ref, src_hbm, x_ref, e_ref, out_ref, tmp_ref, xbf_ref, src_ref, sem_a, sem_b
):
    i = pl.program_id(0)

    @pl.when(i == 0)
    def _():
        pltpu.make_async_copy(
            src_hbm.at[pl.ds(0, NH)], src_ref.at[pl.ds(0, NH)], sem_a
        ).start()
        pltpu.make_async_copy(
            src_hbm.at[pl.ds(NH, NH)], src_ref.at[pl.ds(NH, NH)], sem_b
        ).start()
        xbf_ref[...] = x_ref[...].astype(jnp.bfloat16)

    @pl.when(i < P1)
    def _():
        tmp_ref[pl.ds(i * BK, BK), :] = jnp.dot(
            tar_ref[...].astype(jnp.bfloat16),
            xbf_ref[...],
            preferred_element_type=jnp.float32,
        )

    @pl.when(i == P1)
    def _():
        pltpu.make_async_copy(
            src_hbm.at[pl.ds(0, NH)], src_ref.at[pl.ds(0, NH)], sem_a
        ).wait()
        pltpu.make_async_copy(
            src_hbm.at[pl.ds(NH, NH)], src_ref.at[pl.ds(NH, NH)], sem_b
        ).wait()

    @pl.when(i >= P1)
    def _():
        j = i - P1
        src_blk = src_ref[pl.ds(j * BN, BN), :]
        out_ref[...] = (
            jnp.dot(src_blk[:, : K // 2], tmp_ref[: K // 2, :], preferred_element_type=jnp.float32)
            + jnp.dot(src_blk[:, K // 2 :], tmp_ref[K // 2 :, :], preferred_element_type=jnp.float32)
            + e_ref[...]
            - x_ref[pl.ds(j * BN, BN), :]
        )


def kernel(t, x, HG_poi_src, HG_poi_tar, e):
    del t
    f = pl.pallas_call(
        _fused_body,
        grid=(P1 + N // BN,),
        in_specs=[
            pl.BlockSpec((BK, N), lambda i: (jnp.minimum(i, P1 - 1), 0)),
            pl.BlockSpec(memory_space=pl.ANY),
            pl.BlockSpec((N, D), lambda i: (0, 0)),
            pl.BlockSpec((BN, D), lambda i: (jnp.maximum(i - P1, 0), 0)),
        ],
        out_specs=pl.BlockSpec((BN, D), lambda i: (jnp.maximum(i - P1, 0), 0)),
        out_shape=jax.ShapeDtypeStruct((N, D), jnp.float32),
        scratch_shapes=[
            pltpu.VMEM((K, D), jnp.float32),
            pltpu.VMEM((N, D), jnp.bfloat16),
            pltpu.VMEM((N, K), jnp.float32),
            pltpu.SemaphoreType.DMA,
            pltpu.SemaphoreType.DMA,
        ],
    )(HG_poi_tar, HG_poi_src, x, e)
    return f
